# baseline (speedup 1.0000x reference)
import functools

import jax
import jax.numpy as jnp
from jax import lax
from jax.experimental import pallas as pl
from jax.experimental.pallas import tpu as pltpu
from jax.experimental.pallas import tpu_sc as plsc

D = 64
ROWS = 4096
COLS = 200
B = ROWS * COLS
NC = 1
NS = 16
NW = NC * NS
BPW = B // NW
SUB = 128
CHUNK = 512
NSUB = CHUNK // SUB
NG = BPW // CHUNK
SCALE = 8.0


def _sc_embed(x2d, table):
    mesh = plsc.VectorSubcoreMesh(
        core_axis_name="c", subcore_axis_name="s", num_cores=NC)

    @functools.partial(
        pl.kernel,
        mesh=mesh,
        compiler_params=pltpu.CompilerParams(use_tc_tiling_on_sc=False),
        out_type=jax.ShapeDtypeStruct((B, D), jnp.float32),
        scratch_types=[
            pltpu.VMEM((NSUB, SUB), jnp.int32),
            pltpu.VMEM((CHUNK, D), jnp.float32),
            pltpu.SemaphoreType.DMA,
        ],
    )
    def k(x_hbm, tbl_hbm, out_hbm, idx_v, rows_v, gsem):
        wid = lax.axis_index("s") * NC + lax.axis_index("c")
        base = wid * BPW
        xbase = wid * (BPW // SUB)

        def chunk_body(g, carry):
            row0 = base + g * CHUNK
            pltpu.sync_copy(x_hbm.at[pl.ds(xbase + g * NSUB, NSUB)], idx_v)
            copies = [
                pltpu.async_copy(
                    tbl_hbm.at[idx_v.at[j]],
                    rows_v.at[pl.ds(j * SUB, SUB)],
                    gsem,
                )
                for j in range(NSUB)
            ]
            for cp in copies:
                cp.wait()

            for j in range(NSUB):
                def grp_body(r8, c, j=j):
                    iv = idx_v[j, pl.ds(r8 * 16, 16)]
                    fv = jnp.where(iv == 0, jnp.float32(0.0),
                                   jnp.float32(SCALE))
                    for t in range(16):
                        f = fv[t]
                        row = j * SUB + r8 * 16 + t
                        for q in range(D // 16):
                            sl = pl.ds(q * 16, 16)
                            rows_v[row, sl] = rows_v[row, sl] * f
                    return c

                lax.fori_loop(0, SUB // 16, grp_body, 0)
            pltpu.sync_copy(rows_v, out_hbm.at[pl.ds(row0, CHUNK)])
            return carry

        lax.fori_loop(0, NG, chunk_body, 0)

    return k(x2d, table)


def kernel(x, table):
    x2d = x.reshape(B // SUB, SUB)
    out = _sc_embed(x2d, table)
    return out.reshape(ROWS, COLS, D)


# speedup vs baseline: 1.4909x; 1.4909x over previous
import functools

import jax
import jax.numpy as jnp
from jax import lax
from jax.experimental import pallas as pl
from jax.experimental.pallas import tpu as pltpu
from jax.experimental.pallas import tpu_sc as plsc

D = 64
XR = 4096
COLS = 200
B = XR * COLS
NC = 2
NS = 16
NW = NC * NS
BPW = B // NW
SUB = 128
CHUNK = 512
NSUB = CHUNK // SUB
NG = BPW // CHUNK
SCALE = 8.0


def _sc_embed(xTr, table):
    mesh = plsc.VectorSubcoreMesh(
        core_axis_name="c", subcore_axis_name="s", num_cores=NC)

    @functools.partial(
        pl.kernel,
        mesh=mesh,
        compiler_params=pltpu.CompilerParams(use_tc_tiling_on_sc=False),
        out_type=jax.ShapeDtypeStruct((COLS, XR, D), jnp.float32),
        scratch_types=[
            pltpu.VMEM((2, NSUB, SUB), jnp.int32),
            pltpu.VMEM((2, CHUNK, D), jnp.float32),
            pltpu.SemaphoreType.DMA,
            pltpu.SemaphoreType.DMA,
            pltpu.SemaphoreType.DMA,
            pltpu.SemaphoreType.DMA,
        ],
    )
    def k(xTr_hbm, tbl_hbm, out_hbm, idx_v, rows, g0, g1, o0, o1):
        wid = lax.axis_index("s") * NC + lax.axis_index("c")
        base = wid * BPW
        qbase = wid * (BPW // SUB)
        gsem = (g0, g1)
        osem = (o0, o1)

        def fire_gathers(b, g):
            pltpu.sync_copy(
                xTr_hbm.at[pl.ds(qbase + g * NSUB, NSUB)], idx_v.at[b])
            for j in range(NSUB):
                pltpu.async_copy(
                    tbl_hbm.at[idx_v.at[b, j]],
                    rows.at[b, pl.ds(j * SUB, SUB), :], gsem[b])

        def wait_sem(sem):
            pltpu.make_async_copy(
                out_hbm.at[0, pl.ds(0, CHUNK)], rows.at[0], sem).wait()

        def scale_chunk(b):
            for j in range(NSUB):
                def grp(kk, cc, j=j):
                    iv = idx_v[b, j, pl.ds(kk * 16, 16)]
                    fv = jnp.where(iv == 0, jnp.float32(0.0),
                                   jnp.float32(SCALE))
                    for t in range(16):
                        f = fv[t]
                        rr = j * SUB + kk * 16 + t
                        for q in range(D // 16):
                            sl = pl.ds(q * 16, 16)
                            rows[b, rr, sl] = rows[b, rr, sl] * f
                    return cc

                lax.fori_loop(0, SUB // 16, grp, 0)

        def store_chunk(b, g):
            p0 = base + g * CHUNK
            jj = lax.shift_right_logical(p0, 12)
            ii = lax.bitwise_and(p0, XR - 1)
            pltpu.async_copy(
                rows.at[b], out_hbm.at[jj, pl.ds(ii, CHUNK)], osem[b])

        fire_gathers(0, 0)

        def body(p, carry):
            @pl.when(p > 0)
            def _():
                wait_sem(osem[1])

            fire_gathers(1, 2 * p + 1)

            wait_sem(gsem[0])
            scale_chunk(0)
            store_chunk(0, 2 * p)

            wait_sem(gsem[1])
            scale_chunk(1)
            store_chunk(1, 2 * p + 1)

            @pl.when(p < NG // 2 - 1)
            def _():
                wait_sem(osem[0])
                fire_gathers(0, 2 * p + 2)

            return carry

        lax.fori_loop(0, NG // 2, body, 0)
        wait_sem(osem[0])
        wait_sem(osem[1])

    return k(xTr, table)


def kernel(x, table):
    xTr = jnp.transpose(x).reshape(B // SUB, SUB)
    outT = _sc_embed(xTr, table)
    return jnp.transpose(outT, (1, 0, 2))


# speedup vs baseline: 1.4936x; 1.0018x over previous
import functools

import jax
import jax.numpy as jnp
from jax import lax
from jax.experimental import pallas as pl
from jax.experimental.pallas import tpu as pltpu
from jax.experimental.pallas import tpu_sc as plsc

D = 64
XR = 4096
COLS = 200
B = XR * COLS
NC = 2
NS = 16
NW = NC * NS
BPW = B // NW
SUB = 128
CHUNK = 512
NSUB = CHUNK // SUB
NG = BPW // CHUNK
SCALE = 8.0


def _sc_embed(xTr, table):
    mesh = plsc.VectorSubcoreMesh(
        core_axis_name="c", subcore_axis_name="s", num_cores=NC)

    @functools.partial(
        pl.kernel,
        mesh=mesh,
        compiler_params=pltpu.CompilerParams(use_tc_tiling_on_sc=False),
        out_type=jax.ShapeDtypeStruct((COLS, XR, D), jnp.float32),
        scratch_types=[
            pltpu.VMEM((2, CHUNK), jnp.int32),
            pltpu.VMEM((2, CHUNK, D), jnp.float32),
            pltpu.SemaphoreType.DMA,
            pltpu.SemaphoreType.DMA,
            pltpu.SemaphoreType.DMA,
            pltpu.SemaphoreType.DMA,
        ],
    )
    def k(xTr_hbm, tbl_hbm, out_hbm, idx_v, rows, g0, g1, o0, o1):
        wid = lax.axis_index("s") * NC + lax.axis_index("c")
        base = wid * BPW
        gsem = (g0, g1)
        osem = (o0, o1)

        def fire_gathers(b, g):
            p0 = base + g * CHUNK
            jj = lax.shift_right_logical(p0, 12)
            ii = pl.multiple_of(lax.bitwise_and(p0, XR - 1), CHUNK)
            pltpu.sync_copy(
                xTr_hbm.at[jj, pl.ds(ii, CHUNK)], idx_v.at[b])
            for j in range(NSUB):
                pltpu.async_copy(
                    tbl_hbm.at[idx_v.at[b, pl.ds(j * SUB, SUB)]],
                    rows.at[b, pl.ds(j * SUB, SUB), :], gsem[b])

        def wait_sem(sem):
            pltpu.make_async_copy(
                out_hbm.at[0, pl.ds(0, CHUNK)], rows.at[0], sem).wait()

        def scale_chunk(b):
            for j in range(NSUB):
                def grp(kk, cc, j=j):
                    iv = idx_v[b, pl.ds(j * SUB + kk * 16, 16)]
                    fv = jnp.where(iv == 0, jnp.float32(0.0),
                                   jnp.float32(SCALE))
                    for t in range(16):
                        f = fv[t]
                        rr = j * SUB + kk * 16 + t
                        for q in range(D // 16):
                            sl = pl.ds(q * 16, 16)
                            rows[b, rr, sl] = rows[b, rr, sl] * f
                    return cc

                lax.fori_loop(0, SUB // 16, grp, 0)

        def store_chunk(b, g):
            p0 = base + g * CHUNK
            jj = lax.shift_right_logical(p0, 12)
            ii = pl.multiple_of(lax.bitwise_and(p0, XR - 1), CHUNK)
            pltpu.async_copy(
                rows.at[b], out_hbm.at[jj, pl.ds(ii, CHUNK)], osem[b])

        fire_gathers(0, 0)

        def body(p, carry):
            @pl.when(p > 0)
            def _():
                wait_sem(osem[1])

            fire_gathers(1, 2 * p + 1)

            wait_sem(gsem[0])
            scale_chunk(0)
            store_chunk(0, 2 * p)

            wait_sem(gsem[1])
            scale_chunk(1)
            store_chunk(1, 2 * p + 1)

            @pl.when(p < NG // 2 - 1)
            def _():
                wait_sem(osem[0])
                fire_gathers(0, 2 * p + 2)

            return carry

        lax.fori_loop(0, NG // 2, body, 0)
        wait_sem(osem[0])
        wait_sem(osem[1])

    return k(xTr, table)


def kernel(x, table):
    outT = _sc_embed(jnp.transpose(x), table)
    return jnp.transpose(outT, (1, 0, 2))
